# baseline (speedup 1.0000x reference)
import jax
import jax.numpy as jnp
from jax import lax
from jax.experimental import pallas as pl
from jax.experimental.pallas import tpu as pltpu
from jax.experimental.pallas import tpu_sc as plsc

N_USERS = 25000
N_ITEMS = 25000
D = 64
E = 800000

NTILE = 16
EPT = E // NTILE
C = 50
S = 500
NCH = S // C
NQ = NCH // 5
NS = EPT // S

RB = 40
NRB = N_USERS // RB
RB_ITERS = -(-NRB // NTILE)

THIRD = 1.0 / 3.0

_mesh = plsc.VectorSubcoreMesh(core_axis_name="c", subcore_axis_name="s")


def _zero_wbuf(wbuf):
    def zrow(r, c):
        for d in range(D // 16):
            wbuf[r, pl.ds(d * 16, 16)] = jnp.zeros((16,), jnp.float32)
        return c
    lax.fori_loop(0, RB, zrow, 0)


_GROUPS = ((0, range(16)), (16, range(16)), (32, range(16)),
           (34, range(14, 16)))


def _scale_chunk(p2, cs, bv, gbuf, mbuf):
    for off, lanes in _GROUPS:
        vals16 = bv[p2, cs, pl.ds(off, 16)]
        for j in lanes:
            v = vals16[j]
            e = off + j
            for d in range(D // 16):
                sl = pl.ds(d * 16, 16)
                mbuf[e, sl] = gbuf[e, sl] * v


def _side(sid, src, gsrc, ssrc, vals, out,
          bg, bs, bv, gbufs, mbufs, wbuf,
          acc, isem, gsems, ssems, base_prev):
    rbase = sid * (EPT // C)

    pltpu.async_copy(gsrc.at[pl.ds(rbase, NCH)], bg.at[0], isem)
    pltpu.async_copy(ssrc.at[pl.ds(rbase, NCH)], bs.at[0], isem)
    pltpu.async_copy(vals.at[pl.ds(rbase, NCH)], bv.at[0], isem)

    _zero_wbuf(wbuf)

    def zchunk(k, c):
        chunk = k * NTILE + sid

        @pl.when(chunk < NRB)
        def _():
            pltpu.sync_copy(wbuf, acc.at[pl.ds(chunk * RB, RB)])
        return c
    lax.fori_loop(0, RB_ITERS, zchunk, 0)
    plsc.subcore_barrier()

    pltpu.make_async_copy(gsrc.at[pl.ds(rbase, NCH)], bg.at[0], isem).wait()
    pltpu.make_async_copy(ssrc.at[pl.ds(rbase, NCH)], bs.at[0], isem).wait()
    pltpu.make_async_copy(vals.at[pl.ds(rbase, NCH)], bv.at[0], isem).wait()
    for i in range(5):
        pltpu.async_copy(src.at[bg.at[0, i]], gbufs[i], gsems[i])

    def sbody(s, c):
        p2 = s & 1
        q2 = 1 - p2
        nb = rbase + (s + 1) * NCH

        @pl.when(s + 1 < NS)
        def _():
            pltpu.async_copy(gsrc.at[pl.ds(nb, NCH)], bg.at[q2], isem)
            pltpu.async_copy(ssrc.at[pl.ds(nb, NCH)], bs.at[q2], isem)
            pltpu.async_copy(vals.at[pl.ds(nb, NCH)], bv.at[q2], isem)

        def qbody(q, c2):
            for i in range(5):
                cs = 5 * q + i
                gb = gbufs[i]
                mb = mbufs[i % 3]
                gsem = gsems[i]
                ssem = ssems[i % 3]

                pltpu.make_async_copy(src.at[bg.at[p2, cs]], gb, gsem).wait()
                if i < 3:
                    @pl.when(q > 0)
                    def _(mb=mb, cs=cs, ssem=ssem):
                        pltpu.make_async_copy(
                            mb, acc.at[bs.at[p2, cs - 3]], ssem).wait()
                else:
                    pltpu.make_async_copy(
                        mb, acc.at[bs.at[p2, cs - 3]], ssem).wait()
                _scale_chunk(p2, cs, bv, gb, mb)

                @pl.when(q < NQ - 1)
                def _(gb=gb, cs=cs, gsem=gsem):
                    pltpu.async_copy(src.at[bg.at[p2, cs + 5]], gb, gsem)
                pltpu.async_copy(mb, acc.at[bs.at[p2, cs]], ssem, add=True)
            return c2
        lax.fori_loop(0, NQ, qbody, 0)

        pltpu.make_async_copy(mbufs[1], acc.at[bs.at[p2, NCH - 3]],
                              ssems[1]).wait()
        pltpu.make_async_copy(mbufs[2], acc.at[bs.at[p2, NCH - 2]],
                              ssems[2]).wait()
        pltpu.make_async_copy(mbufs[0], acc.at[bs.at[p2, NCH - 1]],
                              ssems[0]).wait()

        @pl.when(s + 1 < NS)
        def _():
            pltpu.make_async_copy(gsrc.at[pl.ds(nb, NCH)], bg.at[q2],
                                  isem).wait()
            pltpu.make_async_copy(ssrc.at[pl.ds(nb, NCH)], bs.at[q2],
                                  isem).wait()
            pltpu.make_async_copy(vals.at[pl.ds(nb, NCH)], bv.at[q2],
                                  isem).wait()
            for i in range(5):
                pltpu.async_copy(src.at[bg.at[q2, i]], gbufs[i], gsems[i])
        return c
    lax.fori_loop(0, NS, sbody, 0)
    plsc.subcore_barrier()

    def wchunk(k, c):
        chunk = k * NTILE + sid

        @pl.when(chunk < NRB)
        def _():
            rb = chunk * RB
            pltpu.sync_copy(acc.at[pl.ds(rb, RB)], wbuf)
            if base_prev is not None:
                bref, pref, bbuf = base_prev
                pltpu.sync_copy(bref.at[pl.ds(rb, RB)], bbuf)

                def arow(r, c2):
                    for d in range(D // 16):
                        sl = pl.ds(d * 16, 16)
                        wbuf[r, sl] = wbuf[r, sl] + bbuf[r, sl]
                    return c2
                lax.fori_loop(0, RB, arow, 0)
                pltpu.sync_copy(pref.at[pl.ds(rb, RB)], bbuf)

                def prow(r, c2):
                    for d in range(D // 16):
                        sl = pl.ds(d * 16, 16)
                        wbuf[r, sl] = (wbuf[r, sl] + bbuf[r, sl]) * THIRD
                    return c2
                lax.fori_loop(0, RB, prow, 0)
            pltpu.sync_copy(wbuf, out.at[pl.ds(rb, RB)])
        return c
    lax.fori_loop(0, RB_ITERS, wchunk, 0)


def _layer1_body(user_w, item_w, rows, cols, vals, out_zu, out_zi,
                 bg, bs, bv, g0, g1, g2, g3, g4, m0, m1, m2, acc,
                 isem, gs0, gs1, gs2, gs3, gs4, ss0, ss1, ss2):
    cid = lax.axis_index("c")
    sid = lax.axis_index("s")
    gbufs = (g0, g1, g2, g3, g4)
    mbufs = (m0, m1, m2)
    gsems = (gs0, gs1, gs2, gs3, gs4)
    ssems = (ss0, ss1, ss2)
    wbuf = g0.at[pl.ds(0, RB)]

    @pl.when(cid == 0)
    def _():
        _side(sid, item_w, cols, rows, vals, out_zu,
              bg, bs, bv, gbufs, mbufs, wbuf, acc,
              isem, gsems, ssems, None)

    @pl.when(cid == 1)
    def _():
        _side(sid, user_w, rows, cols, vals, out_zi,
              bg, bs, bv, gbufs, mbufs, wbuf, acc,
              isem, gsems, ssems, None)


def _layer2_body(user_w, item_w, zu1, zi1, rows, cols, vals, out_u, out_i,
                 bg, bs, bv, g0, g1, g2, g3, g4, m0, m1, m2, acc,
                 isem, gs0, gs1, gs2, gs3, gs4, ss0, ss1, ss2):
    cid = lax.axis_index("c")
    sid = lax.axis_index("s")
    gbufs = (g0, g1, g2, g3, g4)
    mbufs = (m0, m1, m2)
    gsems = (gs0, gs1, gs2, gs3, gs4)
    ssems = (ss0, ss1, ss2)
    wbuf = g0.at[pl.ds(0, RB)]
    bbuf = g1.at[pl.ds(0, RB)]

    @pl.when(cid == 0)
    def _():
        _side(sid, zi1, cols, rows, vals, out_u,
              bg, bs, bv, gbufs, mbufs, wbuf, acc,
              isem, gsems, ssems, (user_w, zu1, bbuf))

    @pl.when(cid == 1)
    def _():
        _side(sid, zu1, rows, cols, vals, out_i,
              bg, bs, bv, gbufs, mbufs, wbuf, acc,
              isem, gsems, ssems, (item_w, zi1, bbuf))


_f32 = jnp.float32
_emb = jax.ShapeDtypeStruct((N_USERS, D), _f32)

_common_scratch = [
    pltpu.VMEM((2, NCH, C), jnp.int32),
    pltpu.VMEM((2, NCH, C), jnp.int32),
    pltpu.VMEM((2, NCH, C), _f32),
    pltpu.VMEM((C, D), _f32),
    pltpu.VMEM((C, D), _f32),
    pltpu.VMEM((C, D), _f32),
    pltpu.VMEM((C, D), _f32),
    pltpu.VMEM((C, D), _f32),
    pltpu.VMEM((C, D), _f32),
    pltpu.VMEM((C, D), _f32),
    pltpu.VMEM((C, D), _f32),
]

_sems = [pltpu.SemaphoreType.DMA] * 9

_params = pltpu.CompilerParams(use_tc_tiling_on_sc=False)

_layer1 = pl.kernel(
    _layer1_body,
    out_type=(_emb, _emb),
    mesh=_mesh,
    compiler_params=_params,
    scratch_types=_common_scratch + [
        pltpu.VMEM_SHARED((N_USERS, D), _f32),
    ] + _sems,
)

_layer2 = pl.kernel(
    _layer2_body,
    out_type=(_emb, _emb),
    mesh=_mesh,
    compiler_params=_params,
    scratch_types=_common_scratch + [
        pltpu.VMEM_SHARED((N_USERS, D), _f32),
    ] + _sems,
)


def kernel(user_w, item_w, adj_rows, adj_cols, adj_vals,
           image_item_embeds, text_item_embeds,
           image_user_embeds, text_user_embeds):
    rows2 = adj_rows.reshape(E // C, C)
    cols2 = adj_cols.reshape(E // C, C)
    vals2 = adj_vals.reshape(E // C, C)
    zu1, zi1 = _layer1(user_w, item_w, rows2, cols2, vals2)
    return _layer2(user_w, item_w, zu1, zi1, rows2, cols2, vals2)


# speedup vs baseline: 1.0893x; 1.0893x over previous
import jax
import jax.numpy as jnp
from jax import lax
from jax.experimental import pallas as pl
from jax.experimental.pallas import tpu as pltpu
from jax.experimental.pallas import tpu_sc as plsc

N_USERS = 25000
N_ITEMS = 25000
D = 64
E = 800000

NTILE = 16
EPT = E // NTILE
C = 50
S = 1000
NCH = S // C
NQ = NCH // 5
NS = EPT // S

RB = 40
NRB = N_USERS // RB
RB_ITERS = -(-NRB // NTILE)

THIRD = 1.0 / 3.0

_mesh = plsc.VectorSubcoreMesh(core_axis_name="c", subcore_axis_name="s")


def _zero_wbuf(wbuf):
    def zrow(r, c):
        for d in range(D // 16):
            wbuf[r, pl.ds(d * 16, 16)] = jnp.zeros((16,), jnp.float32)
        return c
    lax.fori_loop(0, RB, zrow, 0)


_GROUPS = ((0, range(16)), (16, range(16)), (32, range(16)),
           (34, range(14, 16)))


def _scale_chunk(p2, cs, bv, gbuf, mbuf):
    for off, lanes in _GROUPS:
        vals16 = bv[p2, cs, pl.ds(off, 16)]
        for j in lanes:
            v = vals16[j]
            e = off + j
            for d in range(D // 16):
                sl = pl.ds(d * 16, 16)
                mbuf[e, sl] = gbuf[e, sl] * v


def _side(sid, src, gsrc, ssrc, vals, out,
          bg, bs, bv, gbufs, mbufs, wbuf,
          acc, isem, gsems, ssems, base_prev):
    rbase = sid * (EPT // C)

    pltpu.async_copy(gsrc.at[pl.ds(rbase, NCH)], bg.at[0], isem)
    pltpu.async_copy(ssrc.at[pl.ds(rbase, NCH)], bs.at[0], isem)
    pltpu.async_copy(vals.at[pl.ds(rbase, NCH)], bv.at[0], isem)

    _zero_wbuf(wbuf)

    def zchunk(k, c):
        chunk = k * NTILE + sid

        @pl.when(chunk < NRB)
        def _():
            pltpu.sync_copy(wbuf, acc.at[pl.ds(chunk * RB, RB)])
        return c
    lax.fori_loop(0, RB_ITERS, zchunk, 0)
    plsc.subcore_barrier()

    pltpu.make_async_copy(gsrc.at[pl.ds(rbase, NCH)], bg.at[0], isem).wait()
    pltpu.make_async_copy(ssrc.at[pl.ds(rbase, NCH)], bs.at[0], isem).wait()
    pltpu.make_async_copy(vals.at[pl.ds(rbase, NCH)], bv.at[0], isem).wait()
    for i in range(5):
        pltpu.async_copy(src.at[bg.at[0, i]], gbufs[i], gsems[i])

    def sbody(s, c):
        p2 = s & 1
        q2 = 1 - p2
        nb = rbase + (s + 1) * NCH

        @pl.when(s + 1 < NS)
        def _():
            pltpu.async_copy(gsrc.at[pl.ds(nb, NCH)], bg.at[q2], isem)
            pltpu.async_copy(ssrc.at[pl.ds(nb, NCH)], bs.at[q2], isem)
            pltpu.async_copy(vals.at[pl.ds(nb, NCH)], bv.at[q2], isem)

        def qbody(q, c2):
            for i in range(5):
                cs = 5 * q + i
                gb = gbufs[i]
                mb = mbufs[i & 1]
                gsem = gsems[i]
                ssem = ssems[i & 1]

                pltpu.make_async_copy(src.at[bg.at[p2, cs]], gb, gsem).wait()
                if i < 2:
                    @pl.when(q > 0)
                    def _(mb=mb, cs=cs, ssem=ssem):
                        pltpu.make_async_copy(
                            mb, acc.at[bs.at[p2, cs - 2]], ssem).wait()
                else:
                    pltpu.make_async_copy(
                        mb, acc.at[bs.at[p2, cs - 2]], ssem).wait()
                _scale_chunk(p2, cs, bv, gb, mb)

                @pl.when(q < NQ - 1)
                def _(gb=gb, cs=cs, gsem=gsem):
                    pltpu.async_copy(src.at[bg.at[p2, cs + 5]], gb, gsem)
                pltpu.async_copy(mb, acc.at[bs.at[p2, cs]], ssem, add=True)
            return c2
        lax.fori_loop(0, NQ, qbody, 0)

        pltpu.make_async_copy(mbufs[0], acc.at[bs.at[p2, NCH - 2]],
                              ssems[0]).wait()
        pltpu.make_async_copy(mbufs[1], acc.at[bs.at[p2, NCH - 1]],
                              ssems[1]).wait()


        @pl.when(s + 1 < NS)
        def _():
            pltpu.make_async_copy(gsrc.at[pl.ds(nb, NCH)], bg.at[q2],
                                  isem).wait()
            pltpu.make_async_copy(ssrc.at[pl.ds(nb, NCH)], bs.at[q2],
                                  isem).wait()
            pltpu.make_async_copy(vals.at[pl.ds(nb, NCH)], bv.at[q2],
                                  isem).wait()
            for i in range(5):
                pltpu.async_copy(src.at[bg.at[q2, i]], gbufs[i], gsems[i])
        return c
    lax.fori_loop(0, NS, sbody, 0)
    plsc.subcore_barrier()

    def wchunk(k, c):
        chunk = k * NTILE + sid

        @pl.when(chunk < NRB)
        def _():
            rb = chunk * RB
            pltpu.sync_copy(acc.at[pl.ds(rb, RB)], wbuf)
            if base_prev is not None:
                bref, pref, bbuf = base_prev
                pltpu.sync_copy(bref.at[pl.ds(rb, RB)], bbuf)

                def arow(r, c2):
                    for d in range(D // 16):
                        sl = pl.ds(d * 16, 16)
                        wbuf[r, sl] = wbuf[r, sl] + bbuf[r, sl]
                    return c2
                lax.fori_loop(0, RB, arow, 0)
                pltpu.sync_copy(pref.at[pl.ds(rb, RB)], bbuf)

                def prow(r, c2):
                    for d in range(D // 16):
                        sl = pl.ds(d * 16, 16)
                        wbuf[r, sl] = (wbuf[r, sl] + bbuf[r, sl]) * THIRD
                    return c2
                lax.fori_loop(0, RB, prow, 0)
            pltpu.sync_copy(wbuf, out.at[pl.ds(rb, RB)])
        return c
    lax.fori_loop(0, RB_ITERS, wchunk, 0)


def _layer1_body(user_w, item_w, rows, cols, vals, out_zu, out_zi,
                 bg, bs, bv, g0, g1, g2, g3, g4, m0, m1, acc,
                 isem, gs0, gs1, gs2, gs3, gs4, ss0, ss1):
    cid = lax.axis_index("c")
    sid = lax.axis_index("s")
    gbufs = (g0, g1, g2, g3, g4)
    mbufs = (m0, m1)
    gsems = (gs0, gs1, gs2, gs3, gs4)
    ssems = (ss0, ss1)
    wbuf = g0.at[pl.ds(0, RB)]

    @pl.when(cid == 0)
    def _():
        _side(sid, item_w, cols, rows, vals, out_zu,
              bg, bs, bv, gbufs, mbufs, wbuf, acc,
              isem, gsems, ssems, None)

    @pl.when(cid == 1)
    def _():
        _side(sid, user_w, rows, cols, vals, out_zi,
              bg, bs, bv, gbufs, mbufs, wbuf, acc,
              isem, gsems, ssems, None)


def _layer2_body(user_w, item_w, zu1, zi1, rows, cols, vals, out_u, out_i,
                 bg, bs, bv, g0, g1, g2, g3, g4, m0, m1, acc,
                 isem, gs0, gs1, gs2, gs3, gs4, ss0, ss1):
    cid = lax.axis_index("c")
    sid = lax.axis_index("s")
    gbufs = (g0, g1, g2, g3, g4)
    mbufs = (m0, m1)
    gsems = (gs0, gs1, gs2, gs3, gs4)
    ssems = (ss0, ss1)
    wbuf = g0.at[pl.ds(0, RB)]
    bbuf = g1.at[pl.ds(0, RB)]

    @pl.when(cid == 0)
    def _():
        _side(sid, zi1, cols, rows, vals, out_u,
              bg, bs, bv, gbufs, mbufs, wbuf, acc,
              isem, gsems, ssems, (user_w, zu1, bbuf))

    @pl.when(cid == 1)
    def _():
        _side(sid, zu1, rows, cols, vals, out_i,
              bg, bs, bv, gbufs, mbufs, wbuf, acc,
              isem, gsems, ssems, (item_w, zi1, bbuf))


_f32 = jnp.float32
_emb = jax.ShapeDtypeStruct((N_USERS, D), _f32)

_common_scratch = [
    pltpu.VMEM((2, NCH, C), jnp.int32),
    pltpu.VMEM((2, NCH, C), jnp.int32),
    pltpu.VMEM((2, NCH, C), _f32),
    pltpu.VMEM((C, D), _f32),
    pltpu.VMEM((C, D), _f32),
    pltpu.VMEM((C, D), _f32),
    pltpu.VMEM((C, D), _f32),
    pltpu.VMEM((C, D), _f32),
    pltpu.VMEM((C, D), _f32),
    pltpu.VMEM((C, D), _f32),
]

_sems = [pltpu.SemaphoreType.DMA] * 8

_params = pltpu.CompilerParams(use_tc_tiling_on_sc=False)

_layer1 = pl.kernel(
    _layer1_body,
    out_type=(_emb, _emb),
    mesh=_mesh,
    compiler_params=_params,
    scratch_types=_common_scratch + [
        pltpu.VMEM_SHARED((N_USERS, D), _f32),
    ] + _sems,
)

_layer2 = pl.kernel(
    _layer2_body,
    out_type=(_emb, _emb),
    mesh=_mesh,
    compiler_params=_params,
    scratch_types=_common_scratch + [
        pltpu.VMEM_SHARED((N_USERS, D), _f32),
    ] + _sems,
)


def kernel(user_w, item_w, adj_rows, adj_cols, adj_vals,
           image_item_embeds, text_item_embeds,
           image_user_embeds, text_user_embeds):
    rows2 = adj_rows.reshape(E // C, C)
    cols2 = adj_cols.reshape(E // C, C)
    vals2 = adj_vals.reshape(E // C, C)
    zu1, zi1 = _layer1(user_w, item_w, rows2, cols2, vals2)
    return _layer2(user_w, item_w, zu1, zi1, rows2, cols2, vals2)
